# baseline (speedup 1.0000x reference)
import functools

import jax
import jax.numpy as jnp
from jax.experimental import pallas as pl

_B = 4096
_D = 64
_TM = 256
_MARGIN_NEG = 1.0


def _loss_tile_kernel(emb_blk, emb_all, lbl_all, out_ref):
    i = pl.program_id(0)

    a = emb_blk[:, :]
    e = emb_all[:, :]
    lbl = lbl_all[0, :]
    lbl_a = lbl_all[0, pl.ds(i * _TM, _TM)]

    sq_a = jnp.sum(a * a, axis=1).reshape(_TM, 1)
    sq_e = jnp.sum(e * e, axis=1).reshape(1, _B)
    dot = jax.lax.dot_general(
        a, e, (((1,), (1,)), ((), ())), preferred_element_type=jnp.float32
    )
    dst = sq_a + sq_e - 2.0 * dot

    same = lbl_a.reshape(_TM, 1) == lbl.reshape(1, _B)
    row = i * _TM + jax.lax.broadcasted_iota(jnp.int32, (_TM, _B), 0)
    col = jax.lax.broadcasted_iota(jnp.int32, (_TM, _B), 1)
    not_self = row != col

    pos_valid = same & not_self & (dst > 0.0)
    neg = _MARGIN_NEG - dst
    neg_valid = (~same) & not_self & (neg > 0.0)

    zero = jnp.float32(0.0)
    pos_sum = jnp.sum(jnp.where(pos_valid, dst, zero), axis=1)
    pos_cnt = jnp.sum(pos_valid.astype(jnp.float32), axis=1)
    neg_sum = jnp.sum(jnp.where(neg_valid, neg, zero), axis=1)
    neg_cnt = jnp.sum(neg_valid.astype(jnp.float32), axis=1)

    partial = (
        (jnp.sum(pos_sum / (pos_cnt + 1e-6)) + jnp.sum(neg_sum / (neg_cnt + 1e-6)))
        / jnp.float32(_B)
    ).reshape(1, 1)

    @pl.when(i == 0)
    def _init():
        out_ref[:, :] = partial

    @pl.when(i != 0)
    def _acc():
        out_ref[:, :] += partial


@functools.partial(jax.jit, static_argnames=())
def kernel(embeddings, labels, add_to_mem):
    del add_to_mem
    lbl2d = labels.reshape(1, _B).astype(jnp.int32)
    out = pl.pallas_call(
        _loss_tile_kernel,
        grid=(_B // _TM,),
        in_specs=[
            pl.BlockSpec((_TM, _D), lambda i: (i, 0)),
            pl.BlockSpec((_B, _D), lambda i: (0, 0)),
            pl.BlockSpec((1, _B), lambda i: (0, 0)),
        ],
        out_specs=pl.BlockSpec((1, 1), lambda i: (0, 0)),
        out_shape=jax.ShapeDtypeStruct((1, 1), jnp.float32),
    )(embeddings, embeddings, lbl2d)
    return out[0, 0]


# speedup vs baseline: 1.5360x; 1.5360x over previous
import functools

import jax
import jax.numpy as jnp
from jax.experimental import pallas as pl

_B = 4096
_D = 64
_TM = 256
_MARGIN_NEG = 1.0
_NEG_BIG = -1e30

_DOT1 = (((1,), (1,)), ((), ()))
_DOTV = (((1,), (0,)), ((), ()))


def _loss_tile_kernel(a_ref, e_ref, lbl_ref, ones_ref, out_ref):
    i = pl.program_id(0)

    a = a_ref[:, :]
    e = e_ref[:, :]
    lbl = lbl_ref[0, :]
    lbl_a = lbl_ref[0, pl.ds(i * _TM, _TM)]
    ones = ones_ref[0, :]

    dst = jax.lax.dot_general(a, e, _DOT1, preferred_element_type=jnp.float32)

    row = i * _TM + jax.lax.broadcasted_iota(jnp.int32, (_TM, _B), 0)
    col = jax.lax.broadcasted_iota(jnp.int32, (_TM, _B), 1)
    dstx = jnp.where(row == col, jnp.float32(_NEG_BIG), dst)

    same = lbl_a.reshape(_TM, 1) == lbl.reshape(1, _B)
    t = jnp.where(same, dstx, jnp.float32(_MARGIN_NEG) - dst)
    r = jnp.maximum(t, 0.0)
    c = jnp.where(t > 0.0, jnp.float32(1.0), jnp.float32(0.0))
    rp = jnp.where(same, r, 0.0)
    cp = jnp.where(same, c, 0.0)

    s_r = jax.lax.dot_general(r, ones, _DOTV, preferred_element_type=jnp.float32)
    s_rp = jax.lax.dot_general(rp, ones, _DOTV, preferred_element_type=jnp.float32)
    s_c = jax.lax.dot_general(c, ones, _DOTV, preferred_element_type=jnp.float32)
    s_cp = jax.lax.dot_general(cp, ones, _DOTV, preferred_element_type=jnp.float32)

    pos = s_rp / (s_cp + 1e-6)
    neg = (s_r - s_rp) / ((s_c - s_cp) + 1e-6)
    partial = (jnp.sum(pos + neg) / jnp.float32(_B)).reshape(1, 1)

    @pl.when(i == 0)
    def _init():
        out_ref[:, :] = partial

    @pl.when(i != 0)
    def _acc():
        out_ref[:, :] += partial


@functools.partial(jax.jit, static_argnames=())
def kernel(embeddings, labels, add_to_mem):
    del add_to_mem
    emb = embeddings.astype(jnp.float32)
    sq = jnp.sum(emb * emb, axis=1, keepdims=True)
    one_col = jnp.ones((_B, 1), jnp.float32)
    a_aug = jnp.concatenate([-2.0 * emb, sq, one_col], axis=1)
    e_aug = jnp.concatenate([emb, one_col, sq], axis=1)
    lbl2d = labels.reshape(1, _B).astype(jnp.int32)
    ones_row = jnp.ones((1, _B), jnp.float32)

    out = pl.pallas_call(
        _loss_tile_kernel,
        grid=(_B // _TM,),
        in_specs=[
            pl.BlockSpec((_TM, _D + 2), lambda i: (i, 0)),
            pl.BlockSpec((_B, _D + 2), lambda i: (0, 0)),
            pl.BlockSpec((1, _B), lambda i: (0, 0)),
            pl.BlockSpec((1, _B), lambda i: (0, 0)),
        ],
        out_specs=pl.BlockSpec((1, 1), lambda i: (0, 0)),
        out_shape=jax.ShapeDtypeStruct((1, 1), jnp.float32),
    )(a_aug, e_aug, lbl2d, ones_row)
    return out[0, 0]


# speedup vs baseline: 1.7376x; 1.1312x over previous
import functools

import jax
import jax.numpy as jnp
from jax.experimental import pallas as pl

_B = 4096
_D = 64
_TM = 256
_MARGIN_NEG = 1.0
_NEG_BIG = -1e30

_DOT1 = (((1,), (1,)), ((), ()))
_DOTV = (((1,), (0,)), ((), ()))


def _loss_tile_kernel(a_ref, e_ref, lbl_ref, ones_ref, out_ref):
    i = pl.program_id(0)

    a = a_ref[:, :]
    e = e_ref[:, :]
    lbl = lbl_ref[0, :]
    lbl_a = lbl_ref[0, pl.ds(i * _TM, _TM)]
    ones = ones_ref[0, :]

    dst = jax.lax.dot_general(a, e, _DOT1, preferred_element_type=jnp.float32)

    row = i * _TM + jax.lax.broadcasted_iota(jnp.int32, (_TM, _B), 0)
    col = jax.lax.broadcasted_iota(jnp.int32, (_TM, _B), 1)
    dstx = jnp.where(row == col, jnp.float32(_NEG_BIG), dst)

    same = lbl_a.reshape(_TM, 1) == lbl.reshape(1, _B)
    t = jnp.where(same, dstx, jnp.float32(_MARGIN_NEG) - dst)
    r = jnp.maximum(t, 0.0)
    g = t > 0.0
    gp = g & same
    rp = jnp.where(same, r, 0.0)

    s_r = jnp.sum(r, axis=1)
    s_rp = jnp.sum(rp, axis=1)
    s_c = jnp.sum(g, axis=1).astype(jnp.float32)
    s_cp = jnp.sum(gp, axis=1).astype(jnp.float32)

    pos = s_rp / (s_cp + 1e-6)
    neg = (s_r - s_rp) / ((s_c - s_cp) + 1e-6)
    partial = (jnp.sum(pos + neg) / jnp.float32(_B)).reshape(1, 1)

    @pl.when(i == 0)
    def _init():
        out_ref[:, :] = partial

    @pl.when(i != 0)
    def _acc():
        out_ref[:, :] += partial


@functools.partial(jax.jit, static_argnames=())
def kernel(embeddings, labels, add_to_mem):
    del add_to_mem
    emb = embeddings.astype(jnp.float32)
    sq = jnp.sum(emb * emb, axis=1, keepdims=True)
    one_col = jnp.ones((_B, 1), jnp.float32)
    a_aug = jnp.concatenate([-2.0 * emb, sq, one_col], axis=1)
    e_aug = jnp.concatenate([emb, one_col, sq], axis=1)
    lbl2d = labels.reshape(1, _B).astype(jnp.int32)
    ones_row = jnp.ones((1, _B), jnp.float32)

    out = pl.pallas_call(
        _loss_tile_kernel,
        grid=(_B // _TM,),
        in_specs=[
            pl.BlockSpec((_TM, _D + 2), lambda i: (i, 0)),
            pl.BlockSpec((_B, _D + 2), lambda i: (0, 0)),
            pl.BlockSpec((1, _B), lambda i: (0, 0)),
            pl.BlockSpec((1, _B), lambda i: (0, 0)),
        ],
        out_specs=pl.BlockSpec((1, 1), lambda i: (0, 0)),
        out_shape=jax.ShapeDtypeStruct((1, 1), jnp.float32),
    )(a_aug, e_aug, lbl2d, ones_row)
    return out[0, 0]
